# baseline (speedup 1.0000x reference)
import functools

import jax
import jax.numpy as jnp
import numpy as np
from jax import lax
from jax.experimental import pallas as pl
from jax.experimental.pallas import tpu as pltpu
from jax.experimental.pallas import tpu_sc as plsc

FRAME_RANGE = 5
NUM_POS_FEATS = 32
DIM_METADATA = 1
TEMPERATURE = 10000.0
B = 16384
EMBED = FRAME_RANGE * 4 * NUM_POS_FEATS
NMETA = FRAME_RANGE * DIM_METADATA
OUT_W = EMBED + NMETA
ROWS_PER_BLOCK = 512

_NC, _NS, _L = 2, 16, 16
_NW = _NC * _NS
_M = 1000000
_RNG = 31264
_MPAD = _RNG * _NW
_NQ = B // _NW
_QCH = 128

_mesh = plsc.VectorSubcoreMesh(
    core_axis_name="c", subcore_axis_name="s", num_cores=_NC, num_subcores=_NS)
_scp = pltpu.CompilerParams(
    needs_layout_passes=False, use_tc_tiling_on_sc=False)


def _wid():
    return lax.axis_index("s") * _NC + lax.axis_index("c")


def _posmap_body(w_hbm, pmap_hbm, wv, buf, sem):
    wid = _wid()
    lo = wid * _RNG
    pltpu.async_copy(w_hbm, wv, sem).wait()
    lane = lax.iota(jnp.int32, _L)
    zeros = jnp.zeros((_L,), jnp.int32)
    onehot = jnp.int32(1) << lane

    def memset_it(r, c):
        buf[pl.ds(pl.multiple_of(r * _L, _L), _L)] = zeros
        return c

    lax.fori_loop(0, _RNG // _L, memset_it, 0)

    def scat_it(jc, c):
        wvv = wv[pl.ds(pl.multiple_of(jc * _L, _L), _L)]
        rel = wvv - lo
        valid = (rel >= 0) & (rel < _RNG)
        rel = jnp.where(valid, rel, 0)
        plsc.store_scatter(buf, [rel], (zeros + jc + 1) * 65536, mask=valid)
        plsc.addupdate_scatter(buf, [rel], onehot, mask=valid)
        return c

    lax.fori_loop(0, B // _L, scat_it, 0)
    pltpu.sync_copy(buf, pmap_hbm.at[pl.ds(lo, _RNG)])


@functools.partial(
    pl.kernel,
    out_type=jax.ShapeDtypeStruct((_MPAD,), jnp.int32),
    mesh=_mesh,
    compiler_params=_scp,
    scratch_types=[
        pltpu.VMEM((B,), jnp.int32),
        pltpu.VMEM((_RNG,), jnp.int32),
        pltpu.SemaphoreType.DMA,
    ],
)
def _posmap(w_hbm, pmap_hbm, wv, buf, sem):
    _posmap_body(w_hbm, pmap_hbm, wv, buf, sem)


def _gatherq_body(pmap_hbm, q_hbm, trk_hbm, meta_hbm, det_hbm,
                  posq_hbm, qout_hbm, tA_hbm, tB_hbm, tC_hbm, mA_hbm, mB_hbm,
                  detg_hbm,
                  qv, cmv, posv, pcv, r0v, r1v, r2v, m0v, m1v,
                  tAv, tBv, tCv, mAv, mBv, detv,
                  sem0, sem1, sem2, sem3):
    wid = _wid()
    base = wid * _NQ
    nch = _NQ // _QCH
    pltpu.async_copy(q_hbm.at[pl.ds(base, _NQ)], qv, sem0).wait()
    pos_cps = [
        pltpu.async_copy(
            pmap_hbm.at[qv.at[pl.ds(c * _QCH, _QCH)]],
            cmv.at[pl.ds(c * _QCH, _QCH)], sem0)
        for c in range(nch)
    ]

    def idx_it(c, carry):
        sl = pl.ds(pl.multiple_of(c * _L, _L), _L)
        q5 = qv[sl] * 5
        r0 = q5 >> 1
        r0v[sl] = r0
        r1v[sl] = r0 + 1
        r2v[sl] = r0 + 2
        m0 = q5 >> 3
        m0v[sl] = m0
        m1v[sl] = jnp.minimum(m0 + 1, _M * NMETA // 8 - 1)
        return carry

    lax.fori_loop(0, _NQ // _L, idx_it, 0)
    trk_cps = [
        pltpu.async_copy(
            trk_hbm.at[iv.at[pl.ds(c * _QCH, _QCH)]],
            dst.at[pl.ds(c * _QCH, _QCH)], sem1)
        for c in range(nch)
        for iv, dst in ((r0v, tAv), (r1v, tBv), (r2v, tCv))
    ]
    meta_cps = [
        pltpu.async_copy(
            meta_hbm.at[iv.at[pl.ds(c * _QCH, _QCH)]],
            dst.at[pl.ds(c * _QCH, _QCH)], sem2)
        for c in range(nch)
        for iv, dst in ((m0v, mAv), (m1v, mBv))
    ]
    for cp in pos_cps:
        cp.wait()

    def decode_it(c, carry):
        sl = pl.ds(pl.multiple_of(c * _L, _L), _L)
        pk = cmv[sl]
        jc = (pk >> 16) - 1
        m = pk & 0xFFFF
        mf = m.astype(jnp.float32)
        hb = (lax.bitcast_convert_type(mf, jnp.int32) >> 23) - 127
        pos = jnp.where(jc >= 0, jc * _L + hb, -1)
        posv[sl] = pos
        pcv[sl] = jnp.maximum(pos, 0)
        return carry

    lax.fori_loop(0, _NQ // _L, decode_it, 0)
    det_cps = [
        pltpu.async_copy(
            det_hbm.at[pcv.at[pl.ds(c * _QCH, _QCH)]],
            detv.at[pl.ds(c * _QCH, _QCH)], sem3)
        for c in range(nch)
    ]
    pltpu.sync_copy(posv, posq_hbm.at[pl.ds(base, _NQ)])
    pltpu.sync_copy(qv, qout_hbm.at[pl.ds(base, _NQ)])
    for cp in trk_cps:
        cp.wait()
    pltpu.sync_copy(tAv, tA_hbm.at[pl.ds(base, _NQ)])
    pltpu.sync_copy(tBv, tB_hbm.at[pl.ds(base, _NQ)])
    pltpu.sync_copy(tCv, tC_hbm.at[pl.ds(base, _NQ)])
    for cp in meta_cps:
        cp.wait()
    pltpu.sync_copy(mAv, mA_hbm.at[pl.ds(base, _NQ)])
    pltpu.sync_copy(mBv, mB_hbm.at[pl.ds(base, _NQ)])
    for cp in det_cps:
        cp.wait()
    pltpu.sync_copy(detv, detg_hbm.at[pl.ds(base, _NQ)])


_B8 = jax.ShapeDtypeStruct((B, 8), jnp.float32)


@functools.partial(
    pl.kernel,
    out_type=(
        jax.ShapeDtypeStruct((B,), jnp.int32),
        jax.ShapeDtypeStruct((B,), jnp.int32),
        _B8, _B8, _B8,
        _B8, _B8,
        _B8,
    ),
    mesh=_mesh,
    compiler_params=_scp,
    scratch_types=[
        pltpu.VMEM((_NQ,), jnp.int32),
        pltpu.VMEM((_NQ,), jnp.int32),
        pltpu.VMEM((_NQ,), jnp.int32),
        pltpu.VMEM((_NQ,), jnp.int32),
        pltpu.VMEM((_NQ,), jnp.int32),
        pltpu.VMEM((_NQ,), jnp.int32),
        pltpu.VMEM((_NQ,), jnp.int32),
        pltpu.VMEM((_NQ,), jnp.int32),
        pltpu.VMEM((_NQ,), jnp.int32),
        pltpu.VMEM((_NQ, 8), jnp.float32),
        pltpu.VMEM((_NQ, 8), jnp.float32),
        pltpu.VMEM((_NQ, 8), jnp.float32),
        pltpu.VMEM((_NQ, 8), jnp.float32),
        pltpu.VMEM((_NQ, 8), jnp.float32),
        pltpu.VMEM((_NQ, 8), jnp.float32),
        pltpu.SemaphoreType.DMA,
        pltpu.SemaphoreType.DMA,
        pltpu.SemaphoreType.DMA,
        pltpu.SemaphoreType.DMA,
    ],
)
def _gatherq(pmap_hbm, q_hbm, trk_hbm, meta_hbm, det_hbm,
             posq_hbm, qout_hbm, tA_hbm, tB_hbm, tC_hbm, mA_hbm, mB_hbm,
             detg_hbm,
             qv, cmv, posv, pcv, r0v, r1v, r2v, m0v, m1v,
             tAv, tBv, tCv, mAv, mBv, detv,
             sem0, sem1, sem2, sem3):
    _gatherq_body(pmap_hbm, q_hbm, trk_hbm, meta_hbm, det_hbm,
                  posq_hbm, qout_hbm, tA_hbm, tB_hbm, tC_hbm, mA_hbm, mB_hbm,
                  detg_hbm,
                  qv, cmv, posv, pcv, r0v, r1v, r2v, m0v, m1v,
                  tAv, tBv, tCv, mAv, mBv, detv,
                  sem0, sem1, sem2, sem3)


def _freq_matrices():
    i = np.arange(NUM_POS_FEATS // 2, dtype=np.float64)
    w = 2.0 * np.pi * TEMPERATURE ** (-i / (NUM_POS_FEATS // 2))
    w2 = np.concatenate([w, w])
    ngroups = FRAME_RANGE * 4
    S20 = np.zeros((ngroups, EMBED), dtype=np.float32)
    for g in range(ngroups):
        S20[g, g * 32:(g + 1) * 32] = w2
    S0 = np.zeros((24, EMBED), dtype=np.float32)
    S0[:20] = S20
    S4OFF = np.zeros((24, EMBED), dtype=np.float32)
    S4OFF[4:24] = S20
    SDET = S20.reshape(FRAME_RANGE, 4, EMBED).sum(axis=0).astype(np.float32)
    shift = np.tile(
        np.concatenate([np.zeros(16), np.full(16, np.pi / 2.0)]), ngroups
    ).astype(np.float32)
    METASEL = np.zeros((8 * NMETA, 16), dtype=np.float32)
    for t in range(8):
        for k in range(NMETA):
            METASEL[5 * t + k, t + k] = 1.0
    E1 = np.zeros((1, 8), dtype=np.float32)
    E1[0, 4] = 1.0
    return S0, S4OFF, SDET, shift[:, None], METASEL, E1


_S0, _S4OFF, _SDET, _SHIFT_T, _METASEL, _E1 = _freq_matrices()


def _encode_body(tA_ref, tB_ref, tC_ref, mA_ref, mB_ref, detg_ref,
                 posq_ref, qout_ref,
                 s0_ref, s4_ref, sdet_ref, shift_ref, msel_ref, e1_ref,
                 out_ref):
    sel = posq_ref[...] >= 0
    q = qout_ref[...]
    qodd = (q & 1) == 1
    dn = (((0,), (1,)), ((), ()))
    dn2 = (((1,), (1,)), ((), ()))
    x24 = jnp.concatenate([tA_ref[...], tB_ref[...], tC_ref[...]], axis=1)
    ph0 = lax.dot_general(s0_ref[...], x24, dn,
                          preferred_element_type=jnp.float32)
    ph4 = lax.dot_general(s4_ref[...], x24, dn,
                          preferred_element_type=jnp.float32)
    trk_ph = jnp.where(qodd, ph4, ph0)
    det_ph = lax.dot_general(sdet_ref[...], detg_ref[:, :4], dn,
                             preferred_element_type=jnp.float32)
    phases = jnp.where(sel, det_ph, trk_ph)
    out_ref[:EMBED, :] = jnp.cos(phases - shift_ref[...])
    x16 = jnp.concatenate([mA_ref[...], mB_ref[...]], axis=1)
    msel = lax.dot_general(msel_ref[...], x16, dn2,
                           preferred_element_type=jnp.float32)
    mo = (q * NMETA) & 7
    meta_nat = jnp.zeros((NMETA, mo.shape[1]), jnp.float32)
    for t in range(8):
        meta_nat = jnp.where(mo == t, msel[5 * t:5 * t + NMETA, :], meta_nat)
    conf = lax.dot_general(e1_ref[...], detg_ref[...], dn2,
                           preferred_element_type=jnp.float32)
    out_ref[EMBED:, :] = jnp.where(
        sel, jnp.broadcast_to(conf, meta_nat.shape), meta_nat)


def _sine_encode(tA, tB, tC, mA, mB, detg, posq1r, qout1r):
    n = tA.shape[0]
    grid = n // ROWS_PER_BLOCK
    r8 = pl.BlockSpec((ROWS_PER_BLOCK, 8), lambda i: (i, 0))
    c1 = pl.BlockSpec((1, ROWS_PER_BLOCK), lambda i: (0, i))
    out_t = pl.pallas_call(
        _encode_body,
        grid=(grid,),
        in_specs=[
            r8, r8, r8, r8, r8, r8, c1, c1,
            pl.BlockSpec((24, EMBED), lambda i: (0, 0)),
            pl.BlockSpec((24, EMBED), lambda i: (0, 0)),
            pl.BlockSpec((4, EMBED), lambda i: (0, 0)),
            pl.BlockSpec((EMBED, 1), lambda i: (0, 0)),
            pl.BlockSpec((8 * NMETA, 16), lambda i: (0, 0)),
            pl.BlockSpec((1, 8), lambda i: (0, 0)),
        ],
        out_specs=pl.BlockSpec((OUT_W, ROWS_PER_BLOCK), lambda i: (0, i)),
        out_shape=jax.ShapeDtypeStruct((OUT_W, n), jnp.float32),
    )(tA, tB, tC, mA, mB, detg, posq1r, qout1r,
      jnp.asarray(_S0), jnp.asarray(_S4OFF), jnp.asarray(_SDET),
      jnp.asarray(_SHIFT_T), jnp.asarray(_METASEL), jnp.asarray(_E1))
    return out_t.T


def kernel(tracklets, tracklet_metadata, detections, write_indices, query_indices):
    trk8 = tracklets.reshape(_M * 4 * FRAME_RANGE // 8, 8)
    meta8 = tracklet_metadata.reshape(_M * NMETA // 8, 8)
    det8 = jnp.pad(detections, ((0, 0), (0, 3)))
    pmap = _posmap(write_indices)
    posq, qout, tA, tB, tC, mA, mB, detg = _gatherq(
        pmap, query_indices, trk8, meta8, det8)
    return _sine_encode(tA, tB, tC, mA, mB, detg,
                        posq.reshape(1, B), qout.reshape(1, B))
